# baseline (speedup 1.0000x reference)
import functools

import jax
import jax.numpy as jnp
from jax import lax
from jax.experimental import pallas as pl
from jax.experimental.pallas import tpu as pltpu
from jax.experimental.pallas import tpu_sc as plsc

VOCAB = 100000
D = 300
DP = 384
BATCH = 4096
T = 50
TP = 56
L = 16
NW = 32
SEQ_PER_W = BATCH // NW
NBUF = 4
GRP = 8
NGRP = SEQ_PER_W // GRP
NFULL = D // L
TAIL = D - L


def _awe_body(w_hbm, idx_hbm, out_hbm, idx_v, out_v, red_v, sem_list, rows_list):
    wid = lax.axis_index("s") * 2 + lax.axis_index("c")
    base = wid * SEQ_PER_W

    pltpu.sync_copy(idx_hbm.at[pl.ds(base, SEQ_PER_W)], idx_v)

    def start_gather(s, buf):
        pltpu.make_async_copy(
            w_hbm.at[idx_v.at[s]], rows_list[buf], sem_list[buf]
        ).start()

    def wait_gather(s, buf):
        pltpu.make_async_copy(
            w_hbm.at[idx_v.at[s]], rows_list[buf], sem_list[buf]
        ).wait()

    for b in range(NBUF):
        start_gather(b, b)

    pos_base = lax.iota(jnp.int32, L)

    def group_body(g, carry):
        s0 = g * GRP
        for b in range(GRP):
            s = s0 + b
            buf = b % NBUF
            wait_gather(s, buf)
            rows = rows_list[buf]

            def row_body(t, accs):
                new = [accs[j] + rows[t, pl.ds(j * L, L)] for j in range(NFULL)]
                new.append(accs[NFULL] + rows[t, pl.ds(TAIL, L)])
                return tuple(new)

            zeros = tuple(jnp.zeros((L,), jnp.float32) for _ in range(NFULL + 1))
            accs = lax.fori_loop(0, T, row_body, zeros)

            @pl.when(s + NBUF < SEQ_PER_W)
            def _():
                start_gather(s + NBUF, buf)

            key = jnp.full((L,), jnp.iinfo(jnp.int32).max, jnp.int32)
            for off in (0, 16, 32, 34):
                v = idx_v[s, pl.ds(off, L)]
                key = jnp.minimum(key, v * 64 + (pos_base + off))
            for sh in (8, 4, 2, 1):
                red_v[pl.ds(0, L)] = key
                key = jnp.minimum(key, plsc.load_gather(red_v, [pos_base ^ sh]))
            pos = key % 64
            length = jnp.where(pos > 0, pos, T)
            scale = 1.0 / length.astype(jnp.float32)

            for j in range(NFULL):
                out_v[b, pl.ds(j * L, L)] = accs[j] * scale
            out_v[b, pl.ds(TAIL, L)] = accs[NFULL] * scale

        pltpu.sync_copy(out_v, out_hbm.at[pl.ds(base + s0, GRP)])
        return carry

    lax.fori_loop(0, NGRP, group_body, 0)


def kernel(input, W):
    idx_padded = jnp.pad(input.astype(jnp.int32), ((0, 0), (0, TP - T)))
    w_padded = jnp.pad(W, ((0, 0), (0, DP - D)))
    mesh = plsc.VectorSubcoreMesh(core_axis_name="c", subcore_axis_name="s")
    run = functools.partial(
        pl.kernel,
        out_type=jax.ShapeDtypeStruct((BATCH, D), jnp.float32),
        mesh=mesh,
        compiler_params=pltpu.CompilerParams(needs_layout_passes=False),
        scratch_types=[
            pltpu.VMEM((SEQ_PER_W, TP), jnp.int32),
            pltpu.VMEM((GRP, D), jnp.float32),
            pltpu.VMEM((L,), jnp.int32),
            [pltpu.SemaphoreType.DMA] * NBUF,
            [pltpu.VMEM((TP, DP), jnp.float32)] * NBUF,
        ],
    )

    @run
    def awe(w_hbm, idx_hbm, out_hbm, idx_v, out_v, red_v, sems, rows):
        _awe_body(w_hbm, idx_hbm, out_hbm, idx_v, out_v, red_v, sems, rows)

    return awe(w_padded, idx_padded)


# speedup vs baseline: 2.2798x; 2.2798x over previous
import functools

import jax
import jax.numpy as jnp
from jax import lax
from jax.experimental import pallas as pl
from jax.experimental.pallas import tpu as pltpu
from jax.experimental.pallas import tpu_sc as plsc

VOCAB = 100000
D = 300
DP = 384
BATCH = 4096
T = 50
TP = 56
L = 16
NW = 32
SEQ_PER_W = BATCH // NW
NBUF = 4
GRP = 8
NGRP = SEQ_PER_W // GRP
NFULL = D // L
TAIL = D - L


def _awe_body(w_hbm, idx_hbm, out_hbm, idx_v, out_v, red_v, sem_list, rows_list):
    wid = lax.axis_index("s") * 2 + lax.axis_index("c")
    base = wid * SEQ_PER_W

    pltpu.sync_copy(idx_hbm.at[pl.ds(base, SEQ_PER_W)], idx_v)

    def start_gather(s, buf):
        pltpu.make_async_copy(
            w_hbm.at[idx_v.at[s]], rows_list[buf], sem_list[buf]
        ).start()

    def wait_gather(s, buf):
        pltpu.make_async_copy(
            w_hbm.at[idx_v.at[s]], rows_list[buf], sem_list[buf]
        ).wait()

    for b in range(NBUF):
        start_gather(b, b)

    pos_base = lax.iota(jnp.int32, L)

    def group_body(g, carry):
        s0 = g * GRP
        for b in range(GRP):
            s = s0 + b
            buf = b % NBUF
            wait_gather(s, buf)
            rows = rows_list[buf]

            def row_body(t, accs):
                new = [accs[j] + rows[t, pl.ds(j * L, L)] for j in range(NFULL)]
                new.append(accs[NFULL] + rows[t, pl.ds(TAIL, L)])
                return tuple(new)

            zeros = tuple(jnp.zeros((L,), jnp.float32) for _ in range(NFULL + 1))
            accs = lax.fori_loop(0, T, row_body, zeros)

            @pl.when(s + NBUF < SEQ_PER_W)
            def _():
                start_gather(s + NBUF, buf)

            key = jnp.full((L,), jnp.iinfo(jnp.int32).max, jnp.int32)
            for off in (0, 16, 32, 34):
                v = idx_v[s, pl.ds(off, L)]
                key = jnp.minimum(key, v * 64 + (pos_base + off))
            for sh in (8, 4, 2, 1):
                red_v[pl.ds(0, L)] = key
                key = jnp.minimum(key, plsc.load_gather(red_v, [pos_base ^ sh]))
            pos = key % 64
            length = jnp.where(pos > 0, pos, T)
            scale = 1.0 / length.astype(jnp.float32)

            for j in range(NFULL):
                out_v[b, pl.ds(j * L, L)] = accs[j] * scale
            out_v[b, pl.ds(TAIL, L)] = accs[NFULL] * scale

        pltpu.sync_copy(out_v, out_hbm.at[pl.ds(base + s0, GRP)])
        return carry

    lax.fori_loop(0, NGRP, group_body, 0)


def kernel(input, W):
    inp = input.astype(jnp.int32)
    idx_padded = jnp.concatenate(
        [inp, jnp.broadcast_to(inp[:, :1], (BATCH, TP - T))], axis=1
    )
    w_padded = jnp.pad(W, ((0, 0), (0, DP - D)))
    mesh = plsc.VectorSubcoreMesh(core_axis_name="c", subcore_axis_name="s")
    run = functools.partial(
        pl.kernel,
        out_type=jax.ShapeDtypeStruct((BATCH, D), jnp.float32),
        mesh=mesh,
        compiler_params=pltpu.CompilerParams(needs_layout_passes=False),
        scratch_types=[
            pltpu.VMEM((SEQ_PER_W, TP), jnp.int32),
            pltpu.VMEM((GRP, D), jnp.float32),
            pltpu.VMEM((L,), jnp.int32),
            [pltpu.SemaphoreType.DMA] * NBUF,
            [pltpu.VMEM((TP, DP), jnp.float32)] * NBUF,
        ],
    )

    @run
    def awe(w_hbm, idx_hbm, out_hbm, idx_v, out_v, red_v, sems, rows):
        _awe_body(w_hbm, idx_hbm, out_hbm, idx_v, out_v, red_v, sems, rows)

    return awe(w_padded, idx_padded)


# speedup vs baseline: 4.6010x; 2.0181x over previous
import functools

import jax
import jax.numpy as jnp
from jax import lax
from jax.experimental import pallas as pl
from jax.experimental.pallas import tpu as pltpu
from jax.experimental.pallas import tpu_sc as plsc

VOCAB = 100000
D = 300
C = 128
TAIL_D = D - 2 * C
BATCH = 4096
T = 50
TP = 56
L = 16
NW = 32
SEQ_PER_W = BATCH // NW
NBUF = 4
GRP = 8
NGRP = SEQ_PER_W // GRP
NFULL = D // L
TAIL = D - L


def _awe_body(w_hbm, wt_hbm, idx_hbm, out_hbm, idx_v, out_v, red_v,
              sem_list, ra_list, rb_list, rt_list):
    wid = lax.axis_index("s") * 2 + lax.axis_index("c")
    base = wid * SEQ_PER_W

    wa = w_hbm.at[:, pl.ds(0, C)]
    wb = w_hbm.at[:, pl.ds(C, C)]

    pltpu.sync_copy(idx_hbm.at[pl.ds(base, SEQ_PER_W)], idx_v)

    def gather_descs(s, buf):
        idx = idx_v.at[s]
        return (
            pltpu.make_async_copy(wa.at[idx], ra_list[buf], sem_list[buf]),
            pltpu.make_async_copy(wb.at[idx], rb_list[buf], sem_list[buf]),
            pltpu.make_async_copy(wt_hbm.at[idx], rt_list[buf], sem_list[buf]),
        )

    def start_gather(s, buf):
        for d in gather_descs(s, buf):
            d.start()

    def wait_gather(s, buf):
        for d in gather_descs(s, buf):
            d.wait()

    for b in range(NBUF):
        start_gather(b, b)

    pos_base = lax.iota(jnp.int32, L)

    def group_body(g, carry):
        s0 = g * GRP
        for b in range(GRP):
            s = s0 + b
            buf = b % NBUF
            wait_gather(s, buf)
            ra, rb, rt = ra_list[buf], rb_list[buf], rt_list[buf]

            def row_body(t, accs):
                new = [accs[j] + ra[t, pl.ds(j * L, L)] for j in range(8)]
                new += [accs[8 + j] + rb[t, pl.ds(j * L, L)] for j in range(8)]
                new += [accs[16 + j] + rt[t, pl.ds(j * L, L)] for j in range(2)]
                new.append(accs[18] + rt[t, pl.ds(TAIL - 2 * C, L)])
                return tuple(new)

            zeros = tuple(jnp.zeros((L,), jnp.float32) for _ in range(NFULL + 1))
            accs = lax.fori_loop(0, T, row_body, zeros)

            @pl.when(s + NBUF < SEQ_PER_W)
            def _():
                start_gather(s + NBUF, buf)

            key = jnp.full((L,), jnp.iinfo(jnp.int32).max, jnp.int32)
            for off in (0, 16, 32, 34):
                v = idx_v[s, pl.ds(off, L)]
                key = jnp.minimum(key, v * 64 + (pos_base + off))
            for sh in (8, 4, 2, 1):
                red_v[pl.ds(0, L)] = key
                key = jnp.minimum(key, plsc.load_gather(red_v, [pos_base ^ sh]))
            pos = key % 64
            length = jnp.where(pos > 0, pos, T)
            scale = 1.0 / length.astype(jnp.float32)

            for j in range(NFULL):
                out_v[b, pl.ds(j * L, L)] = accs[j] * scale
            out_v[b, pl.ds(TAIL, L)] = accs[NFULL] * scale

        pltpu.sync_copy(out_v, out_hbm.at[pl.ds(base + s0, GRP)])
        return carry

    lax.fori_loop(0, NGRP, group_body, 0)


def kernel(input, W):
    inp = input.astype(jnp.int32)
    idx_padded = jnp.concatenate(
        [inp, jnp.broadcast_to(inp[:, :1], (BATCH, TP - T))], axis=1
    )
    w_tail = jnp.pad(W[:, 2 * C:], ((0, 0), (0, C - TAIL_D)))

    mesh = plsc.VectorSubcoreMesh(core_axis_name="c", subcore_axis_name="s")
    run = functools.partial(
        pl.kernel,
        out_type=jax.ShapeDtypeStruct((BATCH, D), jnp.float32),
        mesh=mesh,
        compiler_params=pltpu.CompilerParams(needs_layout_passes=False),
        scratch_types=[
            pltpu.VMEM((SEQ_PER_W, TP), jnp.int32),
            pltpu.VMEM((GRP, D), jnp.float32),
            pltpu.VMEM((L,), jnp.int32),
            [pltpu.SemaphoreType.DMA] * NBUF,
            [pltpu.VMEM((TP, C), jnp.float32)] * NBUF,
            [pltpu.VMEM((TP, C), jnp.float32)] * NBUF,
            [pltpu.VMEM((TP, C), jnp.float32)] * NBUF,
        ],
    )

    @run
    def awe(w_hbm, wt_hbm, idx_hbm, out_hbm, idx_v, out_v, red_v,
            sems, ra, rb, rt):
        _awe_body(w_hbm, wt_hbm, idx_hbm, out_hbm, idx_v, out_v, red_v,
                  sems, ra, rb, rt)

    return awe(W, w_tail, idx_padded)
